# baseline (speedup 1.0000x reference)
import functools

import jax
import jax.numpy as jnp
from jax import lax
from jax.experimental import pallas as pl
from jax.experimental.pallas import tpu as pltpu
from jax.experimental.pallas import tpu_sc as plsc

B = 4096
SEQ = 200
D = 64
DP = 16
NC, NS = 2, 16
NW = NC * NS
BPW = B // NW
GRP = 8
NGRP = BPW // GRP
GROWS = GRP * SEQ


VPAD = 102400
SLAB = VPAD // 8
PBLK = 1280


def _proj_body(e0, e1, e2, e3, e4, e5, e6, e7, wpb_ref, p_ref):
    embs = (e0, e1, e2, e3, e4, e5, e6, e7)
    acc = jnp.zeros(p_ref.shape, jnp.float32)
    for a in range(8):
        acc = acc + lax.dot_general(
            embs[a][...], wpb_ref[a * D:(a + 1) * D, :],
            (((0,), (0,)), ((), ())),
            preferred_element_type=jnp.float32,
        )
    p_ref[...] = acc * (1.0 / SEQ)


def _project(embt, wpbig):
    v = embt.shape[1]
    nblk = v // PBLK
    grid = SLAB // PBLK

    def espec(a):
        return pl.BlockSpec(
            (D, PBLK),
            lambda j, a=a: (0, jnp.minimum(a * grid + j, nblk)),
        )

    return pl.pallas_call(
        _proj_body,
        grid=(grid,),
        in_specs=[espec(a) for a in range(8)]
        + [pl.BlockSpec((8 * D, 8 * DP), lambda j: (0, 0))],
        out_specs=pl.BlockSpec((PBLK, 8 * DP), lambda j: (j, 0)),
        out_shape=jax.ShapeDtypeStruct((SLAB, 8 * DP), jnp.float32),
    )(*([embt] * 8), wpbig)


def _make_sc_kernel():
    mesh = plsc.VectorSubcoreMesh(core_axis_name="c", subcore_axis_name="s")

    @functools.partial(
        pl.kernel,
        mesh=mesh,
        compiler_params=pltpu.CompilerParams(
            use_tc_tiling_on_sc=False, needs_layout_passes=False),
        out_type=jax.ShapeDtypeStruct((B, DP), jnp.float32),
        scratch_types=[
            pltpu.VMEM((SEQ, BPW), jnp.int32),
            pltpu.VMEM((BPW * SEQ,), jnp.int32),
            pltpu.VMEM((2, GROWS, DP), jnp.float32),
            pltpu.VMEM((BPW, DP), jnp.float32),
            pltpu.VMEM((DP,), jnp.float32),
            pltpu.SemaphoreType.DMA,
            pltpu.SemaphoreType.DMA,
        ],
    )
    def sc_kernel(textt_hbm, bias_hbm, p_hbm, out_hbm, stage_v, idx_v, rows_v,
                  res_v, bias_v, sem0, sem1):
        sems = (sem0, sem1)
        wid = lax.axis_index("s") * NC + lax.axis_index("c")
        base = wid * BPW
        pltpu.sync_copy(textt_hbm.at[:, pl.ds(base, BPW)], stage_v)
        pltpu.sync_copy(bias_hbm, bias_v)
        bvec = bias_v[...]
        lanes = lax.iota(jnp.int32, 16) * SEQ

        def transform_band(m):
            def step(t, _):
                x = stage_v[t, pl.ds(16 * m, 16)]
                a = lax.shift_right_logical(
                    lax.shift_right_logical(x, 9) * 1311, 15)
                r = ((x - a * SLAB) << 3) | a
                plsc.store_scatter(idx_v, [lanes + (t + 16 * m * SEQ)], r)
                return 0

            lax.fori_loop(0, SEQ, step, 0)

        def fire(g, buf):
            gbase = g * GROWS
            for j in range(12):
                pltpu.async_copy(
                    p_hbm.at[idx_v.at[pl.ds(gbase + 128 * j, 128)]],
                    rows_v.at[buf, pl.ds(128 * j, 128)], sems[buf])
            pltpu.async_copy(
                p_hbm.at[idx_v.at[pl.ds(gbase + 1536, 64)]],
                rows_v.at[buf, pl.ds(1536, 64)], sems[buf])

        def drain(buf):
            pltpu.make_async_copy(
                p_hbm.at[pl.ds(0, GROWS)], rows_v.at[buf], sems[buf]).wait()

        def reduce_group(g, buf):
            zero = jnp.zeros((DP,), jnp.float32)

            def rstep(r, accs):
                return tuple(
                    accs[k] + (rows_v[buf, k * SEQ + 2 * r]
                               + rows_v[buf, k * SEQ + 2 * r + 1])
                    for k in range(GRP))

            accs = lax.fori_loop(0, SEQ // 2, rstep, (zero,) * GRP)
            for k in range(GRP):
                res_v[g * GRP + k] = accs[k] + bvec

        transform_band(0)
        fire(0, 0)
        fire(1, 1)

        def pair(i, _):
            @pl.when(i + 1 < NGRP // 2)
            def _():
                transform_band(i + 1)

            for buf in range(2):
                g = 2 * i + buf
                drain(buf)
                reduce_group(g, buf)

                @pl.when(g + 2 < NGRP)
                def _():
                    fire(g + 2, buf)
            return 0

        lax.fori_loop(0, NGRP // 2, pair, 0)
        pltpu.sync_copy(res_v, out_hbm.at[pl.ds(base, BPW)])

    return sc_kernel


_sc_kernel = _make_sc_kernel()


def kernel(text, offsets, emb, W, b):
    del offsets
    nc = W.shape[0]
    wp = jnp.zeros((D, DP), jnp.float32).at[:, :nc].set(W.T)
    wpbig = jnp.kron(jnp.eye(8, dtype=jnp.float32), wp)
    bp = jnp.zeros((DP,), jnp.float32).at[:nc].set(b)
    p = _project(emb.T, wpbig).reshape(VPAD, DP)
    out16 = _sc_kernel(text.astype(jnp.int32).T, bp, p)
    return out16[:, :nc]


# speedup vs baseline: 1.0109x; 1.0109x over previous
import functools

import jax
import jax.numpy as jnp
from jax import lax
from jax.experimental import pallas as pl
from jax.experimental.pallas import tpu as pltpu
from jax.experimental.pallas import tpu_sc as plsc

B = 4096
SEQ = 200
D = 64
DP = 16
NC, NS = 2, 16
NW = NC * NS
BPW = B // NW
GRP = 8
NGRP = BPW // GRP
GROWS = GRP * SEQ


VPAD = 102400
SLAB = VPAD // 8
PBLK = 1280


def _proj_body(e0, e1, e2, e3, e4, e5, e6, e7, wpb_ref, p_ref):
    embs = (e0, e1, e2, e3, e4, e5, e6, e7)
    acc = jnp.zeros(p_ref.shape, jnp.float32)
    for a in range(8):
        acc = acc + lax.dot_general(
            embs[a][...], wpb_ref[a * D:(a + 1) * D, :],
            (((0,), (0,)), ((), ())),
            preferred_element_type=jnp.float32,
        )
    p_ref[...] = acc * (1.0 / SEQ)


def _project(embt, wpbig):
    v = embt.shape[1]
    nblk = v // PBLK
    grid = SLAB // PBLK

    def espec(a):
        return pl.BlockSpec(
            (D, PBLK),
            lambda j, a=a: (0, jnp.minimum(a * grid + j, nblk)),
        )

    return pl.pallas_call(
        _proj_body,
        grid=(grid,),
        in_specs=[espec(a) for a in range(8)]
        + [pl.BlockSpec((8 * D, 8 * DP), lambda j: (0, 0))],
        out_specs=pl.BlockSpec((PBLK, 8 * DP), lambda j: (j, 0)),
        out_shape=jax.ShapeDtypeStruct((SLAB, 8 * DP), jnp.float32),
    )(*([embt] * 8), wpbig)


def _make_sc_kernel():
    mesh = plsc.VectorSubcoreMesh(core_axis_name="c", subcore_axis_name="s")

    @functools.partial(
        pl.kernel,
        mesh=mesh,
        compiler_params=pltpu.CompilerParams(
            use_tc_tiling_on_sc=False, needs_layout_passes=False),
        out_type=jax.ShapeDtypeStruct((DP, B), jnp.float32),
        scratch_types=[
            pltpu.VMEM((SEQ, BPW), jnp.int32),
            pltpu.VMEM((BPW * SEQ,), jnp.int32),
            pltpu.VMEM((2, GROWS, DP), jnp.float32),
            pltpu.VMEM((DP, BPW), jnp.float32),
            pltpu.VMEM((DP,), jnp.float32),
            pltpu.SemaphoreType.DMA,
            pltpu.SemaphoreType.DMA,
        ],
    )
    def sc_kernel(textt_hbm, bias_hbm, p_hbm, out_hbm, stage_v, idx_v, rows_v,
                  res_v, bias_v, sem0, sem1):
        sems = (sem0, sem1)
        wid = lax.axis_index("s") * NC + lax.axis_index("c")
        base = wid * BPW
        pltpu.sync_copy(textt_hbm.at[:, pl.ds(base, BPW)], stage_v)
        pltpu.sync_copy(bias_hbm, bias_v)
        bvec = bias_v[...]
        lanes = lax.iota(jnp.int32, 16) * SEQ
        clanes = lax.iota(jnp.int32, 16)

        def transform_band(m):
            def step(t, _):
                x = stage_v[t, pl.ds(16 * m, 16)]
                a = lax.shift_right_logical(
                    lax.shift_right_logical(x, 9) * 1311, 15)
                r = ((x - a * SLAB) << 3) | a
                plsc.store_scatter(idx_v, [lanes + (t + 16 * m * SEQ)], r)
                return 0

            lax.fori_loop(0, SEQ, step, 0)

        def fire(g, buf):
            gbase = g * GROWS
            for j in range(12):
                pltpu.async_copy(
                    p_hbm.at[idx_v.at[pl.ds(gbase + 128 * j, 128)]],
                    rows_v.at[buf, pl.ds(128 * j, 128)], sems[buf])
            pltpu.async_copy(
                p_hbm.at[idx_v.at[pl.ds(gbase + 1536, 64)]],
                rows_v.at[buf, pl.ds(1536, 64)], sems[buf])

        def drain(buf):
            pltpu.make_async_copy(
                p_hbm.at[pl.ds(0, GROWS)], rows_v.at[buf], sems[buf]).wait()

        def reduce_group(g, buf):
            zero = jnp.zeros((DP,), jnp.float32)

            def rstep(r, accs):
                return tuple(
                    accs[k] + (rows_v[buf, k * SEQ + 2 * r]
                               + rows_v[buf, k * SEQ + 2 * r + 1])
                    for k in range(GRP))

            accs = lax.fori_loop(0, SEQ // 2, rstep, (zero,) * GRP)
            for k in range(GRP):
                plsc.store_scatter(
                    res_v, [clanes, jnp.full((DP,), g * GRP + k, jnp.int32)],
                    accs[k] + bvec)

        transform_band(0)
        fire(0, 0)
        fire(1, 1)

        def pair(i, _):
            @pl.when(i + 1 < NGRP // 2)
            def _():
                transform_band(i + 1)

            for buf in range(2):
                g = 2 * i + buf
                drain(buf)
                reduce_group(g, buf)

                @pl.when(g + 2 < NGRP)
                def _():
                    fire(g + 2, buf)
            return 0

        lax.fori_loop(0, NGRP // 2, pair, 0)
        pltpu.sync_copy(res_v, out_hbm.at[:, pl.ds(base, BPW)])

    return sc_kernel


_sc_kernel = _make_sc_kernel()


def kernel(text, offsets, emb, W, b):
    del offsets
    nc = W.shape[0]
    wp = jnp.zeros((D, DP), jnp.float32).at[:, :nc].set(W.T)
    wpbig = jnp.kron(jnp.eye(8, dtype=jnp.float32), wp)
    bp = jnp.zeros((DP,), jnp.float32).at[:nc].set(b)
    p = _project(emb.T, wpbig).reshape(VPAD, DP)
    out16t = _sc_kernel(text.astype(jnp.int32).T, bp, p)
    return out16t[:nc, :].T


# speedup vs baseline: 1.0657x; 1.0542x over previous
import functools

import jax
import jax.numpy as jnp
from jax import lax
from jax.experimental import pallas as pl
from jax.experimental.pallas import tpu as pltpu
from jax.experimental.pallas import tpu_sc as plsc

B = 4096
SEQ = 200
D = 64
DP = 16
NC, NS = 2, 16
NW = NC * NS
BPW = B // NW
GRP = 8
NGRP = BPW // GRP
GROWS = GRP * SEQ


VPAD = 102400
SLAB = VPAD // 8
PBLK = 1280


def _proj_body(e0, e1, e2, e3, e4, e5, e6, e7, wpb_ref, p_ref):
    embs = (e0, e1, e2, e3, e4, e5, e6, e7)
    acc = jnp.zeros(p_ref.shape, jnp.float32)
    for a in range(8):
        acc = acc + lax.dot_general(
            embs[a][...], wpb_ref[a * D:(a + 1) * D, :],
            (((0,), (0,)), ((), ())),
            preferred_element_type=jnp.float32,
        )
    p_ref[...] = acc * (1.0 / SEQ)


def _project(embt, wpbig):
    v = embt.shape[1]
    nblk = v // PBLK
    grid = SLAB // PBLK

    def espec(a):
        return pl.BlockSpec(
            (D, PBLK),
            lambda j, a=a: (0, jnp.minimum(a * grid + j, nblk)),
        )

    return pl.pallas_call(
        _proj_body,
        grid=(grid,),
        in_specs=[espec(a) for a in range(8)]
        + [pl.BlockSpec((8 * D, 8 * DP), lambda j: (0, 0))],
        out_specs=pl.BlockSpec((PBLK, 8 * DP), lambda j: (j, 0)),
        out_shape=jax.ShapeDtypeStruct((SLAB, 8 * DP), jnp.float32),
    )(*([embt] * 8), wpbig)


def _make_sc_kernel():
    mesh = plsc.VectorSubcoreMesh(core_axis_name="c", subcore_axis_name="s")

    @functools.partial(
        pl.kernel,
        mesh=mesh,
        compiler_params=pltpu.CompilerParams(
            use_tc_tiling_on_sc=False, needs_layout_passes=False),
        out_type=jax.ShapeDtypeStruct((DP, B), jnp.float32),
        scratch_types=[
            pltpu.VMEM((SEQ // 8, 8, BPW), jnp.int32),
            pltpu.VMEM((BPW * SEQ,), jnp.int32),
            pltpu.VMEM((2, GROWS, DP), jnp.float32),
            pltpu.VMEM((DP, BPW), jnp.float32),
            pltpu.VMEM((DP,), jnp.float32),
            pltpu.SemaphoreType.DMA,
            pltpu.SemaphoreType.DMA,
        ],
    )
    def sc_kernel(textt_hbm, bias_hbm, p_hbm, out_hbm, stage_v, idx_v, rows_v,
                  res_v, bias_v, sem0, sem1):
        sems = (sem0, sem1)
        wid = lax.axis_index("s") * NC + lax.axis_index("c")
        base = wid * BPW
        pltpu.sync_copy(textt_hbm.at[:, wid], stage_v)
        pltpu.sync_copy(bias_hbm, bias_v)
        bvec = bias_v[...]
        lanes = lax.iota(jnp.int32, 16) * SEQ
        clanes = lax.iota(jnp.int32, 16)

        def transform_band(m):
            def step(t, _):
                x = stage_v[t >> 3, t & 7, pl.ds(16 * m, 16)]
                a = lax.shift_right_logical(
                    lax.shift_right_logical(x, 9) * 1311, 15)
                r = ((x - a * SLAB) << 3) | a
                plsc.store_scatter(idx_v, [lanes + (t + 16 * m * SEQ)], r)
                return 0

            lax.fori_loop(0, SEQ, step, 0)

        def fire(g, buf):
            gbase = g * GROWS
            for j in range(12):
                pltpu.async_copy(
                    p_hbm.at[idx_v.at[pl.ds(gbase + 128 * j, 128)]],
                    rows_v.at[buf, pl.ds(128 * j, 128)], sems[buf])
            pltpu.async_copy(
                p_hbm.at[idx_v.at[pl.ds(gbase + 1536, 64)]],
                rows_v.at[buf, pl.ds(1536, 64)], sems[buf])

        def drain(buf):
            pltpu.make_async_copy(
                p_hbm.at[pl.ds(0, GROWS)], rows_v.at[buf], sems[buf]).wait()

        def reduce_group(g, buf):
            zero = jnp.zeros((DP,), jnp.float32)

            def rstep(r, accs):
                return tuple(
                    accs[k] + (rows_v[buf, k * SEQ + 2 * r]
                               + rows_v[buf, k * SEQ + 2 * r + 1])
                    for k in range(GRP))

            accs = lax.fori_loop(0, SEQ // 2, rstep, (zero,) * GRP)
            for k in range(GRP):
                plsc.store_scatter(
                    res_v, [clanes, jnp.full((DP,), g * GRP + k, jnp.int32)],
                    accs[k] + bvec)

        transform_band(0)
        fire(0, 0)
        fire(1, 1)

        def pair(i, _):
            @pl.when(i + 1 < NGRP // 2)
            def _():
                transform_band(i + 1)

            for buf in range(2):
                g = 2 * i + buf
                drain(buf)
                reduce_group(g, buf)

                @pl.when(g + 2 < NGRP)
                def _():
                    fire(g + 2, buf)
            return 0

        lax.fori_loop(0, NGRP // 2, pair, 0)
        pltpu.sync_copy(res_v, out_hbm.at[:, pl.ds(base, BPW)])

    return sc_kernel


_sc_kernel = _make_sc_kernel()


def kernel(text, offsets, emb, W, b):
    del offsets
    nc = W.shape[0]
    wp = jnp.zeros((D, DP), jnp.float32).at[:, :nc].set(W.T)
    wpbig = jnp.kron(jnp.eye(8, dtype=jnp.float32), wp)
    bp = jnp.zeros((DP,), jnp.float32).at[:nc].set(b)
    p = _project(emb.T, wpbig).reshape(VPAD, DP)
    t4 = (text.astype(jnp.int32).T
          .reshape(SEQ // 8, 8, B // 128, 128)
          .transpose(0, 2, 1, 3))
    out16t = _sc_kernel(t4, bp, p)
    return out16t[:nc, :].T


# speedup vs baseline: 1.0877x; 1.0206x over previous
import functools

import jax
import jax.numpy as jnp
from jax import lax
from jax.experimental import pallas as pl
from jax.experimental.pallas import tpu as pltpu
from jax.experimental.pallas import tpu_sc as plsc

B = 4096
SEQ = 200
D = 64
DP = 16
NC, NS = 2, 16
NW = NC * NS
BPW = B // NW
GRP = 8
NGRP = BPW // GRP
GROWS = GRP * SEQ


VPAD = 102400
SLAB = VPAD // 8
PBLK = 2560


def _proj_body(e0, e1, e2, e3, e4, e5, e6, e7, wpb_ref, p_ref):
    embs = (e0, e1, e2, e3, e4, e5, e6, e7)
    acc = jnp.zeros(p_ref.shape, jnp.float32)
    for a in range(8):
        acc = acc + lax.dot_general(
            embs[a][...], wpb_ref[a * D:(a + 1) * D, :],
            (((0,), (0,)), ((), ())),
            preferred_element_type=jnp.float32,
        )
    p_ref[...] = acc * (1.0 / SEQ)


def _project(embt, wpbig):
    v = embt.shape[1]
    nblk = v // PBLK
    grid = SLAB // PBLK

    def espec(a):
        return pl.BlockSpec(
            (D, PBLK),
            lambda j, a=a: (0, jnp.minimum(a * grid + j, nblk)),
        )

    return pl.pallas_call(
        _proj_body,
        grid=(grid,),
        in_specs=[espec(a) for a in range(8)]
        + [pl.BlockSpec((8 * D, 8 * DP), lambda j: (0, 0))],
        out_specs=pl.BlockSpec((PBLK, 8 * DP), lambda j: (j, 0)),
        out_shape=jax.ShapeDtypeStruct((SLAB, 8 * DP), jnp.float32),
    )(*([embt] * 8), wpbig)


def _make_sc_kernel():
    mesh = plsc.VectorSubcoreMesh(core_axis_name="c", subcore_axis_name="s")

    @functools.partial(
        pl.kernel,
        mesh=mesh,
        compiler_params=pltpu.CompilerParams(
            use_tc_tiling_on_sc=False, needs_layout_passes=False),
        out_type=jax.ShapeDtypeStruct((DP, B), jnp.float32),
        scratch_types=[
            pltpu.VMEM((SEQ // 8, 8, BPW), jnp.int32),
            pltpu.VMEM((BPW * SEQ,), jnp.int32),
            pltpu.VMEM((2, GROWS, DP), jnp.float32),
            pltpu.VMEM((DP, BPW), jnp.float32),
            pltpu.VMEM((DP,), jnp.float32),
            pltpu.SemaphoreType.DMA,
            pltpu.SemaphoreType.DMA,
        ],
    )
    def sc_kernel(textt_hbm, bias_hbm, p_hbm, out_hbm, stage_v, idx_v, rows_v,
                  res_v, bias_v, sem0, sem1):
        sems = (sem0, sem1)
        wid = lax.axis_index("s") * NC + lax.axis_index("c")
        base = wid * BPW
        pltpu.sync_copy(textt_hbm.at[:, wid], stage_v)
        pltpu.sync_copy(bias_hbm, bias_v)
        bvec = bias_v[...]
        lanes = lax.iota(jnp.int32, 16) * SEQ
        clanes = lax.iota(jnp.int32, 16)

        def transform_band(m):
            def step(t, _):
                x = stage_v[t >> 3, t & 7, pl.ds(16 * m, 16)]
                a = lax.shift_right_logical(
                    lax.shift_right_logical(x, 9) * 1311, 15)
                r = ((x - a * SLAB) << 3) | a
                plsc.store_scatter(idx_v, [lanes + (t + 16 * m * SEQ)], r)
                return 0

            lax.fori_loop(0, SEQ, step, 0)

        def fire(g, buf):
            gbase = g * GROWS
            for j in range(12):
                pltpu.async_copy(
                    p_hbm.at[idx_v.at[pl.ds(gbase + 128 * j, 128)]],
                    rows_v.at[buf, pl.ds(128 * j, 128)], sems[buf])
            pltpu.async_copy(
                p_hbm.at[idx_v.at[pl.ds(gbase + 1536, 64)]],
                rows_v.at[buf, pl.ds(1536, 64)], sems[buf])

        def drain(buf):
            pltpu.make_async_copy(
                p_hbm.at[pl.ds(0, GROWS)], rows_v.at[buf], sems[buf]).wait()

        def reduce_group(g, buf):
            zero = jnp.zeros((DP,), jnp.float32)

            def rstep(r, accs):
                return tuple(
                    accs[k] + (rows_v[buf, k * SEQ + 2 * r]
                               + rows_v[buf, k * SEQ + 2 * r + 1])
                    for k in range(GRP))

            accs = lax.fori_loop(0, SEQ // 2, rstep, (zero,) * GRP)
            for k in range(GRP):
                plsc.store_scatter(
                    res_v, [clanes, jnp.full((DP,), g * GRP + k, jnp.int32)],
                    accs[k] + bvec)

        transform_band(0)
        fire(0, 0)
        fire(1, 1)

        def pair(i, _):
            @pl.when(i + 1 < NGRP // 2)
            def _():
                transform_band(i + 1)

            for buf in range(2):
                g = 2 * i + buf
                drain(buf)
                reduce_group(g, buf)

                @pl.when(g + 2 < NGRP)
                def _():
                    fire(g + 2, buf)
            return 0

        lax.fori_loop(0, NGRP // 2, pair, 0)
        pltpu.sync_copy(res_v, out_hbm.at[:, pl.ds(base, BPW)])

    return sc_kernel


_sc_kernel = _make_sc_kernel()


def kernel(text, offsets, emb, W, b):
    del offsets
    nc = W.shape[0]
    wp = jnp.zeros((D, DP), jnp.float32).at[:, :nc].set(W.T)
    wpbig = jnp.kron(jnp.eye(8, dtype=jnp.float32), wp)
    bp = jnp.zeros((DP,), jnp.float32).at[:nc].set(b)
    p = _project(emb.T, wpbig).reshape(VPAD, DP)
    t4 = (text.astype(jnp.int32).T
          .reshape(SEQ // 8, 8, B // 128, 128)
          .transpose(0, 2, 1, 3))
    out16t = _sc_kernel(t4, bp, p)
    return out16t[:nc, :].T
